# Initial kernel scaffold; baseline (speedup 1.0000x reference)
import jax
import jax.numpy as jnp
from jax.experimental import pallas as pl


def kernel(node_attrs, node_feats, edge_attrs, edge_feats, edge_index, W_up, W_lin, W_skip):
    raise NotImplementedError("write your pallas kernel here")


# speedup vs baseline: 1.7716x; 1.7716x over previous
import functools

import jax
import jax.numpy as jnp
import numpy as np
from jax import lax
from jax.experimental import pallas as pl
from jax.experimental.pallas import tpu as pltpu

_N = 10000
_E = 160000
_C = 128
_NLM = 9
_NL = 3
_NELEM = 10
_AVG = 16.0
_LMAP = (0, 1, 1, 1, 2, 2, 2, 2, 2)
_GROUPS = ((0, 1), (1, 3), (4, 5))
_NPAD = 10240


def _prep_kernel(nf_ref, wup_ref, wlin_ref, wskip_ref, h_ref, wcomb_ref):
    h_ref[...] = jnp.dot(nf_ref[...], wup_ref[...],
                         preferred_element_type=jnp.float32)
    for l in range(_NL):
        a = wlin_ref[l] * (1.0 / _AVG)
        b = wskip_ref[:, l]
        wcomb_ref[:, l] = jax.vmap(
            lambda bz: jnp.dot(a, bz, preferred_element_type=jnp.float32))(b)


def _prep(node_feats, W_up, W_lin, W_skip):
    return pl.pallas_call(
        _prep_kernel,
        out_shape=(
            jax.ShapeDtypeStruct((_N, _C), jnp.float32),
            jax.ShapeDtypeStruct((_NELEM, _NL, _C, _C), jnp.float32),
        ),
    )(node_feats, W_up, W_lin, W_skip)


_ECHUNK = 2000


def _msg_kernel(send_ref, recv_ref, ea_ref, ef_ref, h_ref, acc_ref):
    step = pl.program_id(0)

    @pl.when(step == 0)
    def _zero():
        acc_ref[...] = jnp.zeros_like(acc_ref)

    lmap = _LMAP

    def body(j, _):
        s = send_ref[0, 0, j]
        r = recv_ref[0, 0, j]
        hrow = h_ref[pl.ds(s, 1), :]
        ef3 = ef_ref[pl.ds(j, 1), :].reshape(_NL, _C)
        t = hrow * ef3
        ea = ea_ref[pl.ds(j, 1), :].reshape(_NLM, 1)
        texp = jnp.concatenate([t[lmap[k]:lmap[k] + 1] for k in range(_NLM)],
                               axis=0)
        m = (ea * texp).reshape(1, _NLM * _C)
        acc_ref[pl.ds(r, 1), :] += m
        return 0

    lax.fori_loop(0, _ECHUNK, body, 0)


def _message_tc(sender3, recv3, edge_attrs, edge_feats, h):
    nsteps = _E // _ECHUNK
    return pl.pallas_call(
        _msg_kernel,
        grid=(nsteps,),
        in_specs=[
            pl.BlockSpec((1, 1, _ECHUNK), lambda i: (i, 0, 0),
                         memory_space=pltpu.SMEM),
            pl.BlockSpec((1, 1, _ECHUNK), lambda i: (i, 0, 0),
                         memory_space=pltpu.SMEM),
            pl.BlockSpec((_ECHUNK, _NLM), lambda i: (i, 0)),
            pl.BlockSpec((_ECHUNK, _NL * _C), lambda i: (i, 0)),
            pl.BlockSpec((_N, _C), lambda i: (0, 0)),
        ],
        out_specs=pl.BlockSpec((_NPAD, _NLM * _C), lambda i: (0, 0)),
        out_shape=jax.ShapeDtypeStruct((_NPAD, _NLM * _C), jnp.float32),
    )(sender3, recv3, edge_attrs, edge_feats, h)


_BN = 512


def _out_kernel(na_ref, msg_ref, wflat_ref, out_ref):
    na = na_ref[...]
    msg = msg_ref[...].reshape(_BN, _NLM, _C)
    for l, (k0, nk) in enumerate(_GROUPS):
        y = (na[:, None, :, None] * msg[:, k0:k0 + nk, None, :])
        y = y.reshape(_BN * nk, _NELEM * _C)
        res = jnp.dot(y, wflat_ref[l], preferred_element_type=jnp.float32)
        out_ref[:, k0:k0 + nk, :] = res.reshape(_BN, nk, _C)


def _skip_contract(node_attrs_pad, msg, wflat):
    nblk = _NPAD // _BN
    return pl.pallas_call(
        _out_kernel,
        grid=(nblk,),
        in_specs=[
            pl.BlockSpec((_BN, _NELEM), lambda i: (i, 0)),
            pl.BlockSpec((_BN, _NLM * _C), lambda i: (i, 0)),
            pl.BlockSpec((_NL, _NELEM * _C, _C), lambda i: (0, 0, 0)),
        ],
        out_specs=pl.BlockSpec((_BN, _NLM, _C), lambda i: (i, 0, 0)),
        out_shape=jax.ShapeDtypeStruct((_NPAD, _NLM, _C), jnp.float32),
    )(node_attrs_pad, msg, wflat)


def kernel(node_attrs, node_feats, edge_attrs, edge_feats, edge_index,
           W_up, W_lin, W_skip):
    h, wcomb = _prep(node_feats, W_up, W_lin, W_skip)

    sender3 = edge_index[0].reshape(_E // _ECHUNK, 1, _ECHUNK)
    recv3 = edge_index[1].reshape(_E // _ECHUNK, 1, _ECHUNK)
    msg = _message_tc(sender3, recv3, edge_attrs, edge_feats, h)

    na_pad = jnp.pad(node_attrs, ((0, _NPAD - _N), (0, 0)))
    wflat = wcomb.transpose(1, 0, 2, 3).reshape(_NL, _NELEM * _C, _C)
    out = _skip_contract(na_pad, msg, wflat)
    return out[:_N]
